# baseline (speedup 1.0000x reference)
import functools

import jax
import jax.numpy as jnp
from jax import lax
from jax.experimental import pallas as pl
from jax.experimental.pallas import tpu as pltpu
from jax.experimental.pallas import tpu_sc as plsc

_NUM_CORES = 2
_NUM_SUBCORES = 16
_NW = _NUM_CORES * _NUM_SUBCORES
_L = 16
_CAP = 256
_LIST = _CAP + 16


@functools.lru_cache(maxsize=None)
def _make_gather_call(B, K, G):
    assert B % (_NW * _L) == 0 and K % _L == 0
    ncols_total = G // 128
    cpw = (ncols_total + _NW - 1) // _NW
    out_rows = B + _NW

    mesh = plsc.VectorSubcoreMesh(
        core_axis_name="c", subcore_axis_name="s",
        num_cores=_NUM_CORES, num_subcores=_NUM_SUBCORES)

    def body(rows_hbm, cols_hbm, ut_hbm, gd_hbm, u_out, g_out,
             idsr_v, idsc_v, lidsu_v, lpidu_v, lidsg_v, lpidg_v, stage_v,
             blk0_v, blk1_v, blk2_v, blk3_v,
             bufu_v, bufg_v, pidxu_v, pidxg_v,
             cnt_s, sem_in, sem_b0, sem_b1, sem_b2, sem_b3, sem_sc):
        wid = lax.axis_index("s") * _NUM_CORES + lax.axis_index("c")
        lo_col = wid * cpw
        ncols = jnp.minimum(cpw, jnp.maximum(ncols_total - lo_col, 0))
        lo_id = lo_col * 128
        hi_id = (lo_col + ncols) * 128
        lanes = lax.iota(jnp.int32, _L)
        blks = (blk0_v, blk1_v, blk2_v, blk3_v)
        sems = (sem_b0, sem_b1, sem_b2, sem_b3)

        pltpu.sync_copy(rows_hbm, idsr_v)
        pltpu.sync_copy(cols_hbm, idsc_v)

        def select_phase(ids_v, lids_v, lpid_v, ci):
            cnt_s[ci] = 0
            def select(v, _):
                ids16 = ids_v[pl.ds(v * _L, _L)]
                m = (ids16 >= lo_id) & (ids16 < hi_id)
                n = cnt_s[ci]
                plsc.store_compressed(lids_v.at[pl.ds(n, _L)], ids16, mask=m)
                plsc.store_compressed(lpid_v.at[pl.ds(n, _L)],
                                      lanes + (v * _L), mask=m)
                cnt_s[ci] = n + plsc.all_reduce_population_count(m)[0]
                return 0
            lax.fori_loop(0, B // _L, select, 0)
            nsel = cnt_s[ci]
            lids_v[pl.ds(nsel, _L)] = jnp.full((_L,), jnp.int32(0x40000000))
            return nsel

        def init_pidx(pidx_v):
            for t in range(_CAP // _L):
                pidx_v[t // 8, pl.ds((t % 8) * _L, _L)] = jnp.full(
                    (_L,), B + wid, jnp.int32)

        def blk_copy(mat_hbm, jj, slot):
            base = (lo_col + jj) * 128
            return pltpu.make_async_copy(
                mat_hbm.at[:, pl.ds(base, 128)], blks[slot], sems[slot])

        def prime(mat_hbm):
            for p in range(4):
                @pl.when(ncols > p)
                def _(p=p):
                    blk_copy(mat_hbm, p, p).start()

        def stream_phase(mat_hbm, lids_v, lpid_v, nsel, buf_v, pidx_v):
            cnt_s[6] = 0

            def process_block(jj, slot):
                blk_copy(mat_hbm, jj, slot).wait()
                blk = blks[slot]
                j = lo_col + jj

                cnt_s[7] = 0
                def scan(v, _):
                    ids16 = lids_v[pl.ds(v * _L, _L)]
                    m = (ids16 >> 7) == j
                    s = cnt_s[7]
                    plsc.store_compressed(stage_v.at[pl.ds(s, _L)], ids16, mask=m)
                    plsc.store_compressed(
                        stage_v.at[pl.ds(64 + s, _L)],
                        lpid_v[pl.ds(v * _L, _L)], mask=m)
                    cnt_s[7] = s + plsc.all_reduce_population_count(m)[0]
                    return 0
                lax.fori_loop(0, (nsel + _L - 1) // _L, scan, 0)
                scnt = cnt_s[7]
                ocnt = cnt_s[6]

                def emit(sg):
                    ids16 = stage_v[pl.ds(sg * _L, _L)]
                    pids16 = stage_v[pl.ds(64 + sg * _L, _L)]
                    m = (lanes + sg * _L) < scnt
                    cmod = ids16 & 127
                    slot16 = ocnt + sg * _L + lanes
                    plsc.store_scatter(
                        pidx_v, [slot16 >> 7, slot16 & 127], pids16, mask=m)
                    for k in range(K):
                        kv = jnp.full((_L,), k, jnp.int32)
                        vals = plsc.load_gather(blk, [kv, cmod])
                        plsc.store_scatter(buf_v, [slot16, kv], vals, mask=m)

                emit(0)
                @pl.when(scnt > _L)
                def _():
                    emit(1)
                cnt_s[6] = ocnt + scnt
                @pl.when(jj + 4 < ncols)
                def _():
                    blk_copy(mat_hbm, jj + 4, slot).start()

            def process_quad(h, _):
                for s in range(4):
                    @pl.when(h * 4 + s < ncols)
                    def _(s=s):
                        process_block(h * 4 + s, s)
                return 0
            lax.fori_loop(0, (ncols + 3) // 4, process_quad, 0)

        def start_scatters(buf_v, pidx_v, mat_out):
            cps = [pltpu.make_async_copy(
                       buf_v.at[pl.ds(t * 128, 128)],
                       mat_out.at[pidx_v.at[t]], sem_sc)
                   for t in range(_CAP // 128)]
            for cp in cps:
                cp.start()
            return cps

        init_pidx(pidxu_v)
        init_pidx(pidxg_v)
        nsel_u = select_phase(idsr_v, lidsu_v, lpidu_v, 0)
        prime(ut_hbm)
        nsel_g = select_phase(idsc_v, lidsg_v, lpidg_v, 1)
        stream_phase(ut_hbm, lidsu_v, lpidu_v, nsel_u, bufu_v, pidxu_v)
        cps_u = start_scatters(bufu_v, pidxu_v, u_out)
        prime(gd_hbm)
        stream_phase(gd_hbm, lidsg_v, lpidg_v, nsel_g, bufg_v, pidxg_v)
        cps_g = start_scatters(bufg_v, pidxg_v, g_out)
        for cp in cps_u + cps_g:
            cp.wait()

    return pl.kernel(
        body,
        out_type=(jax.ShapeDtypeStruct((out_rows, 128), jnp.float32),
                  jax.ShapeDtypeStruct((out_rows, 128), jnp.float32)),
        mesh=mesh,
        scratch_types=[
            pltpu.VMEM((B,), jnp.int32),
            pltpu.VMEM((B,), jnp.int32),
            pltpu.VMEM((_LIST,), jnp.int32),
            pltpu.VMEM((_LIST,), jnp.int32),
            pltpu.VMEM((_LIST,), jnp.int32),
            pltpu.VMEM((_LIST,), jnp.int32),
            pltpu.VMEM((128,), jnp.int32),
            pltpu.VMEM((K, 128), jnp.float32),
            pltpu.VMEM((K, 128), jnp.float32),
            pltpu.VMEM((K, 128), jnp.float32),
            pltpu.VMEM((K, 128), jnp.float32),
            pltpu.VMEM((_CAP, 128), jnp.float32),
            pltpu.VMEM((_CAP, 128), jnp.float32),
            pltpu.VMEM((_CAP // 128, 128), jnp.int32),
            pltpu.VMEM((_CAP // 128, 128), jnp.int32),
            pltpu.SMEM((8,), jnp.int32),
            pltpu.SemaphoreType.DMA,
            pltpu.SemaphoreType.DMA,
            pltpu.SemaphoreType.DMA,
            pltpu.SemaphoreType.DMA,
            pltpu.SemaphoreType.DMA,
            pltpu.SemaphoreType.DMA,
        ],
        compiler_params=pltpu.CompilerParams(
            needs_layout_passes=False, use_tc_tiling_on_sc=True),
        name="als_gather_sc",
    )


def _dot_kernel(u_ref, g_ref, r_ref, c_ref, ut_ref, gt_ref, o_ref, *, K, TB, TW):
    u = u_ref[:, :K]
    g = g_ref[:, :K]
    r = r_ref[...][:, None]
    c = c_ref[...][:, None]
    if TW:
        sel = lax.broadcasted_iota(jnp.int32, (1, TW), 1)
        u_fix = jnp.dot(jnp.where(r - TB == sel, 1.0, 0.0), ut_ref[...],
                        preferred_element_type=jnp.float32,
                        precision=lax.Precision.HIGHEST)
        g_fix = jnp.dot(jnp.where(c - TB == sel, 1.0, 0.0), gt_ref[...],
                        preferred_element_type=jnp.float32,
                        precision=lax.Precision.HIGHEST)
        u = jnp.where(r >= TB, u_fix, u)
        g = jnp.where(c >= TB, g_fix, g)
    o_ref[...] = jnp.sum(u * g, axis=1)


@functools.lru_cache(maxsize=None)
def _make_dot_call(B, K, TB, TW):
    blk = 2048
    return pl.pallas_call(
        functools.partial(_dot_kernel, K=K, TB=TB, TW=TW),
        grid=(B // blk,),
        in_specs=[pl.BlockSpec((blk, 128), lambda i: (i, 0)),
                  pl.BlockSpec((blk, 128), lambda i: (i, 0)),
                  pl.BlockSpec((blk,), lambda i: (i,)),
                  pl.BlockSpec((blk,), lambda i: (i,)),
                  pl.BlockSpec((TW or 1, K), lambda i: (0, 0)),
                  pl.BlockSpec((TW or 1, K), lambda i: (0, 0))],
        out_specs=pl.BlockSpec((blk,), lambda i: (i,)),
        out_shape=jax.ShapeDtypeStruct((B,), jnp.float32),
        name="als_pair_dot_tc",
    )


def kernel(location, user_matrix, goods_matrix):
    B = location.shape[0]
    U, K = user_matrix.shape
    _, G = goods_matrix.shape
    rows = location[:, 0].astype(jnp.int32)
    cols = location[:, 1].astype(jnp.int32)
    user_t = user_matrix.T
    u_gath, g_gath = _make_gather_call(B, K, G)(rows, cols, user_t, goods_matrix)
    tb = (G // 128) * 128
    tw = G - tb
    u_tail = user_matrix[tb:G, :] if tw else user_matrix[:1, :]
    g_tail = goods_matrix[:, tb:G].T if tw else goods_matrix[:, :1].T
    out = _make_dot_call(B, K, tb, tw)(
        u_gath, g_gath, rows, cols, u_tail, g_tail)
    return out[:, None]


# speedup vs baseline: 1.0507x; 1.0507x over previous
import functools

import jax
import jax.numpy as jnp
from jax import lax
from jax.experimental import pallas as pl
from jax.experimental.pallas import tpu as pltpu
from jax.experimental.pallas import tpu_sc as plsc

_NUM_CORES = 2
_NUM_SUBCORES = 16
_NW = _NUM_CORES * _NUM_SUBCORES
_L = 16
_CAP = 256
_LIST = _CAP + 16


@functools.lru_cache(maxsize=None)
def _make_gather_call(B, K, G):
    assert B % (_NW * _L) == 0 and K % _L == 0
    ncols_total = G // 128
    cpw = (ncols_total + _NW - 1) // _NW
    out_rows = B + _NW

    mesh = plsc.VectorSubcoreMesh(
        core_axis_name="c", subcore_axis_name="s",
        num_cores=_NUM_CORES, num_subcores=_NUM_SUBCORES)

    def body(rows_hbm, cols_hbm, ut_hbm, gd_hbm, u_out, g_out,
             idsr_v, idsc_v, lidsu_v, lpidu_v, lidsg_v, lpidg_v, stage_v,
             blk0_v, blk1_v, blk2_v, blk3_v,
             bufu_v, bufg_v, pidxu_v, pidxg_v,
             cnt_s, sem_in, sem_b0, sem_b1, sem_b2, sem_b3, sem_sc):
        wid = lax.axis_index("s") * _NUM_CORES + lax.axis_index("c")
        lo_col = wid * cpw
        ncols = jnp.minimum(cpw, jnp.maximum(ncols_total - lo_col, 0))
        lo_id = lo_col * 128
        hi_id = (lo_col + ncols) * 128
        lanes = lax.iota(jnp.int32, _L)
        blks = (blk0_v, blk1_v, blk2_v, blk3_v)
        sems = (sem_b0, sem_b1, sem_b2, sem_b3)

        pltpu.sync_copy(rows_hbm, idsr_v)
        pltpu.sync_copy(cols_hbm, idsc_v)

        def select_phase(ids_v, lids_v, lpid_v, ci):
            cnt_s[ci] = 0
            def select(v, _):
                ids16 = ids_v[pl.ds(v * _L, _L)]
                m = (ids16 >= lo_id) & (ids16 < hi_id)
                n = cnt_s[ci]
                plsc.store_compressed(lids_v.at[pl.ds(n, _L)], ids16, mask=m)
                plsc.store_compressed(lpid_v.at[pl.ds(n, _L)],
                                      lanes + (v * _L), mask=m)
                cnt_s[ci] = n + plsc.all_reduce_population_count(m)[0]
                return 0
            lax.fori_loop(0, B // _L, select, 0)
            nsel = cnt_s[ci]
            lids_v[pl.ds(nsel, _L)] = jnp.full((_L,), jnp.int32(0x40000000))
            return nsel

        def init_pidx(pidx_v):
            for t in range(_CAP // _L):
                pidx_v[t // 8, pl.ds((t % 8) * _L, _L)] = jnp.full(
                    (_L,), B + wid, jnp.int32)

        def blk_copy(mat_hbm, jj, slot):
            base = (lo_col + jj) * 128
            return pltpu.make_async_copy(
                mat_hbm.at[:, pl.ds(base, 128)], blks[slot], sems[slot])

        def prime(mat_hbm):
            for p in range(4):
                @pl.when(ncols > p)
                def _(p=p):
                    blk_copy(mat_hbm, p, p).start()

        def stream_phase(mat_hbm, lids_v, lpid_v, nsel, buf_v, pidx_v):
            cnt_s[6] = 0

            def process_block(jj, slot):
                blk_copy(mat_hbm, jj, slot).wait()
                blk = blks[slot]
                j = lo_col + jj

                cnt_s[7] = 0
                def scan(v, _):
                    ids16 = lids_v[pl.ds(v * _L, _L)]
                    m = (ids16 >> 7) == j
                    s = cnt_s[7]
                    plsc.store_compressed(stage_v.at[pl.ds(s, _L)], ids16, mask=m)
                    plsc.store_compressed(
                        stage_v.at[pl.ds(64 + s, _L)],
                        lpid_v[pl.ds(v * _L, _L)], mask=m)
                    cnt_s[7] = s + plsc.all_reduce_population_count(m)[0]
                    return 0
                lax.fori_loop(0, (nsel + _L - 1) // _L, scan, 0)
                scnt = cnt_s[7]
                ocnt = cnt_s[6]

                def emit(sg):
                    ids16 = stage_v[pl.ds(sg * _L, _L)]
                    pids16 = stage_v[pl.ds(64 + sg * _L, _L)]
                    m = (lanes + sg * _L) < scnt
                    cmod = ids16 & 127
                    slot16 = ocnt + sg * _L + lanes
                    plsc.store_scatter(
                        pidx_v, [slot16 >> 7, slot16 & 127], pids16, mask=m)
                    def kloop(t, _):
                        for q in range(8):
                            kv = jnp.full((_L,), t * 8 + q, jnp.int32)
                            vals = plsc.load_gather(blk, [kv, cmod])
                            plsc.store_scatter(buf_v, [slot16, kv], vals, mask=m)
                        return 0
                    lax.fori_loop(0, K // 8, kloop, 0)

                emit(0)
                @pl.when(scnt > _L)
                def _():
                    emit(1)
                cnt_s[6] = ocnt + scnt
                @pl.when(jj + 4 < ncols)
                def _():
                    blk_copy(mat_hbm, jj + 4, slot).start()

            def process_quad(h, _):
                for s in range(4):
                    @pl.when(h * 4 + s < ncols)
                    def _(s=s):
                        process_block(h * 4 + s, s)
                return 0
            lax.fori_loop(0, (ncols + 3) // 4, process_quad, 0)

        def start_scatters(buf_v, pidx_v, mat_out):
            cps = [pltpu.make_async_copy(
                       buf_v.at[pl.ds(t * 128, 128)],
                       mat_out.at[pidx_v.at[t]], sem_sc)
                   for t in range(_CAP // 128)]
            for cp in cps:
                cp.start()
            return cps

        init_pidx(pidxu_v)
        init_pidx(pidxg_v)
        nsel_u = select_phase(idsr_v, lidsu_v, lpidu_v, 0)
        prime(ut_hbm)
        nsel_g = select_phase(idsc_v, lidsg_v, lpidg_v, 1)
        stream_phase(ut_hbm, lidsu_v, lpidu_v, nsel_u, bufu_v, pidxu_v)
        cps_u = start_scatters(bufu_v, pidxu_v, u_out)
        prime(gd_hbm)
        stream_phase(gd_hbm, lidsg_v, lpidg_v, nsel_g, bufg_v, pidxg_v)
        cps_g = start_scatters(bufg_v, pidxg_v, g_out)
        for cp in cps_u + cps_g:
            cp.wait()

    return pl.kernel(
        body,
        out_type=(jax.ShapeDtypeStruct((out_rows, 128), jnp.float32),
                  jax.ShapeDtypeStruct((out_rows, 128), jnp.float32)),
        mesh=mesh,
        scratch_types=[
            pltpu.VMEM((B,), jnp.int32),
            pltpu.VMEM((B,), jnp.int32),
            pltpu.VMEM((_LIST,), jnp.int32),
            pltpu.VMEM((_LIST,), jnp.int32),
            pltpu.VMEM((_LIST,), jnp.int32),
            pltpu.VMEM((_LIST,), jnp.int32),
            pltpu.VMEM((128,), jnp.int32),
            pltpu.VMEM((K, 128), jnp.float32),
            pltpu.VMEM((K, 128), jnp.float32),
            pltpu.VMEM((K, 128), jnp.float32),
            pltpu.VMEM((K, 128), jnp.float32),
            pltpu.VMEM((_CAP, 128), jnp.float32),
            pltpu.VMEM((_CAP, 128), jnp.float32),
            pltpu.VMEM((_CAP // 128, 128), jnp.int32),
            pltpu.VMEM((_CAP // 128, 128), jnp.int32),
            pltpu.SMEM((8,), jnp.int32),
            pltpu.SemaphoreType.DMA,
            pltpu.SemaphoreType.DMA,
            pltpu.SemaphoreType.DMA,
            pltpu.SemaphoreType.DMA,
            pltpu.SemaphoreType.DMA,
            pltpu.SemaphoreType.DMA,
        ],
        compiler_params=pltpu.CompilerParams(
            needs_layout_passes=False, use_tc_tiling_on_sc=True),
        name="als_gather_sc",
    )


def _dot_kernel(u_ref, g_ref, r_ref, c_ref, ut_ref, gt_ref, o_ref, *, K, TB, TW):
    u = u_ref[:, :K]
    g = g_ref[:, :K]
    r = r_ref[...][:, None]
    c = c_ref[...][:, None]
    if TW:
        sel = lax.broadcasted_iota(jnp.int32, (1, TW), 1)
        u_fix = jnp.dot(jnp.where(r - TB == sel, 1.0, 0.0), ut_ref[...],
                        preferred_element_type=jnp.float32,
                        precision=lax.Precision.HIGHEST)
        g_fix = jnp.dot(jnp.where(c - TB == sel, 1.0, 0.0), gt_ref[...],
                        preferred_element_type=jnp.float32,
                        precision=lax.Precision.HIGHEST)
        u = jnp.where(r >= TB, u_fix, u)
        g = jnp.where(c >= TB, g_fix, g)
    o_ref[...] = jnp.sum(u * g, axis=1)


@functools.lru_cache(maxsize=None)
def _make_dot_call(B, K, TB, TW):
    blk = 2048
    return pl.pallas_call(
        functools.partial(_dot_kernel, K=K, TB=TB, TW=TW),
        grid=(B // blk,),
        in_specs=[pl.BlockSpec((blk, 128), lambda i: (i, 0)),
                  pl.BlockSpec((blk, 128), lambda i: (i, 0)),
                  pl.BlockSpec((blk,), lambda i: (i,)),
                  pl.BlockSpec((blk,), lambda i: (i,)),
                  pl.BlockSpec((TW or 1, K), lambda i: (0, 0)),
                  pl.BlockSpec((TW or 1, K), lambda i: (0, 0))],
        out_specs=pl.BlockSpec((blk,), lambda i: (i,)),
        out_shape=jax.ShapeDtypeStruct((B,), jnp.float32),
        name="als_pair_dot_tc",
    )


def kernel(location, user_matrix, goods_matrix):
    B = location.shape[0]
    U, K = user_matrix.shape
    _, G = goods_matrix.shape
    rows = location[:, 0].astype(jnp.int32)
    cols = location[:, 1].astype(jnp.int32)
    user_t = user_matrix.T
    u_gath, g_gath = _make_gather_call(B, K, G)(rows, cols, user_t, goods_matrix)
    tb = (G // 128) * 128
    tw = G - tb
    u_tail = user_matrix[tb:G, :] if tw else user_matrix[:1, :]
    g_tail = goods_matrix[:, tb:G].T if tw else goods_matrix[:, :1].T
    out = _make_dot_call(B, K, tb, tw)(
        u_gath, g_gath, rows, cols, u_tail, g_tail)
    return out[:, None]
